# Initial kernel scaffold; baseline (speedup 1.0000x reference)
import jax
import jax.numpy as jnp
from jax.experimental import pallas as pl


def kernel(x, emb, w, b):
    raise NotImplementedError("write your pallas kernel here")


# speedup vs baseline: 250.8425x; 250.8425x over previous
import functools

import jax
import jax.numpy as jnp
from jax.experimental import pallas as pl
from jax.experimental.pallas import tpu as pltpu

_V = 30000


def _round_up(x, m):
    return ((x + m - 1) // m) * m


def _pool_head_kernel(ids_ref, emb_ref, w_ref, b_ref, o_ref, pooled_ref,
                      *, tb, s):
    base = pl.program_id(0) * tb
    for r in range(tb):
        acc = emb_ref[ids_ref[base + r, 0], 0]
        for t in range(1, s):
            acc = acc + emb_ref[ids_ref[base + r, t], 0]
        pooled_ref[r, :] = acc
    logits = jnp.dot(pooled_ref[...], w_ref[...],
                     preferred_element_type=jnp.float32)
    o_ref[...] = logits + b_ref[...]


def kernel(x, emb, w, b):
    B, S = x.shape
    Vr, Hp = emb.shape
    Cp = w.shape[1]
    tb = 8
    Bp = _round_up(B, tb)

    ids = x.astype(jnp.int32)
    ids = jnp.where((ids >= 0) & (ids < _V), ids, _V)
    ids = jnp.pad(ids, ((0, Bp - B), (0, 0)), constant_values=_V)

    emb3 = emb.reshape(Vr, 1, Hp)
    w_scaled = w * jnp.float32(1.0 / S)

    out = pl.pallas_call(
        functools.partial(_pool_head_kernel, tb=tb, s=S),
        out_shape=jax.ShapeDtypeStruct((Bp, Cp), jnp.float32),
        grid_spec=pltpu.PrefetchScalarGridSpec(
            num_scalar_prefetch=1,
            grid=(Bp // tb,),
            in_specs=[
                pl.BlockSpec((Vr, 1, Hp), lambda i, ids: (0, 0, 0),
                             pipeline_mode=pl.Buffered(1)),
                pl.BlockSpec((Hp, Cp), lambda i, ids: (0, 0),
                             pipeline_mode=pl.Buffered(1)),
                pl.BlockSpec((1, Cp), lambda i, ids: (0, 0),
                             pipeline_mode=pl.Buffered(1)),
            ],
            out_specs=pl.BlockSpec((tb, Cp), lambda i, ids: (i, 0)),
            scratch_shapes=[pltpu.VMEM((tb, Hp), jnp.float32)],
        ),
        compiler_params=pltpu.CompilerParams(
            dimension_semantics=("parallel",),
            vmem_limit_bytes=48 * 1024 * 1024,
        ),
    )(ids, emb3, w_scaled, b)
    return out[:B, :Cp]
